# Initial kernel scaffold; baseline (speedup 1.0000x reference)
import jax
import jax.numpy as jnp
from jax.experimental import pallas as pl


def kernel(x, pos, batch, x_skip, pos_skip, batch_skip, edge_index, global_token, Wq, bq, Wk, bk, Wv, bv, in_proj_w, in_proj_b, out_proj_w, out_proj_b, mlp_W, mlp_b, bn_g, bn_b, root_W, conv_b):
    raise NotImplementedError("write your pallas kernel here")


# speedup vs baseline: 2.2687x; 2.2687x over previous
import numpy as np
import jax
import jax.numpy as jnp
from jax import lax
from jax.experimental import pallas as pl
from jax.experimental.pallas import tpu as pltpu
from jax.experimental.pallas import tpu_sc as plsc

N_C, N_S, E = 2500, 10000, 160000
EMB, HEADS, DH, T = 64, 4, 16, 64
NCP = 2560
TS = 400
TE = 4000
CHUNK = 128
NCHK = E // CHUNK
NWORK = 32
CPW = 40
EPAD = NWORK * CPW * CHUNK

_F32 = jnp.float32


def _make_pe(L, d):
    position = np.arange(L, dtype=np.float32)[:, None]
    div = np.exp(np.arange(0, d, 2).astype(np.float32) * (-np.log(10000.0) / d))
    pe = np.zeros((L, d), dtype=np.float32)
    pe[:, 0::2] = np.sin(position * div)
    pe[:, 1::2] = np.cos(position * div)
    return pe


_PE = _make_pe(T, EMB)
_PERM = np.zeros((256, 256), np.float32)
for _o in range(16):
    for _i in range(16):
        _PERM[_o * 16 + _i, _i * 16 + _o] = 1.0
_RT = np.zeros((16, 256), np.float32)
for _o in range(16):
    for _i in range(16):
        _RT[_i, _o * 16 + _i] = 1.0
_RM = np.zeros((256, 16), np.float32)
for _j in range(256):
    _RM[_j, _j // 16] = 1.0


def _knn_body(ps_ref, posT_ref, xpad_ref, xs_ref, out_ref):
    n = ps_ref.shape[0]
    d2 = jnp.zeros((n, NCP), _F32)
    for c in range(3):
        a = ps_ref[:, c:c + 1]
        b = posT_ref[c:c + 1, :]
        d2 = d2 + (a - b) ** 2
    lanes = lax.broadcasted_iota(jnp.int32, (n, NCP), 1)
    wmat = jnp.zeros((n, NCP), _F32)
    wsum = jnp.zeros((n, 1), _F32)
    dd = d2
    for _ in range(3):
        i_r = jnp.argmin(dd, axis=1)[:, None]
        m_r = jnp.min(dd, axis=1)[:, None]
        w_r = 1.0 / jnp.maximum(m_r, 1e-16)
        sel = lanes == i_r
        wmat = wmat + jnp.where(sel, w_r, 0.0)
        wsum = wsum + w_r
        dd = jnp.where(sel, jnp.inf, dd)
    xu = jnp.dot(wmat, xpad_ref[...], preferred_element_type=_F32) / wsum
    out_ref[...] = jnp.concatenate([xu, xs_ref[...]], axis=1)


def _prologue_body(gt_ref, wk_ref, wv_ref, ip_ref, wqp_ref, wo_ref, bias_ref,
                   pe_ref, kT_ref, v_ref, qwT_ref, cons_ref, woT_ref):
    gt = gt_ref[...]
    bq = bias_ref[0, :][None, :]
    bk = bias_ref[1, :][None, :]
    bv = bias_ref[2, :][None, :]
    biq = bias_ref[3, :][None, :]
    bik = bias_ref[4, :][None, :]
    biv = bias_ref[5, :][None, :]
    ob = bias_ref[6, :][None, :]
    pe = pe_ref[...]
    ct = lambda a, b: lax.dot_general(a, b, (((1,), (1,)), ((), ())),
                                      preferred_element_type=_F32)
    K = ct(gt, wk_ref[...]) + bk + pe
    V = ct(gt, wv_ref[...]) + bv + pe
    wiq = ip_ref[0:EMB]
    wik = ip_ref[EMB:2 * EMB]
    wiv = ip_ref[2 * EMB:3 * EMB]
    kproj = ct(K, wik) + bik
    vproj = ct(V, wiv) + biv
    kT_ref[...] = kproj.T
    v_ref[...] = vproj
    qw = jnp.dot(wiq, wqp_ref[...], preferred_element_type=_F32)
    qwT_ref[...] = qw.T
    qb = ct(bq, wiq) + biq
    cons = jnp.concatenate(
        [qb, ob, jnp.zeros((6, EMB), _F32)], axis=0)
    cons_ref[...] = cons
    woT_ref[...] = wo_ref[...].T


def _attn_body(g0_ref, g1_ref, kT_ref, v_ref, qwT_ref, cons_ref, woT_ref,
               gtok_ref, sp_ref, s1_ref, s2_ref):
    step = pl.program_id(0)
    n = g0_ref.shape[0]
    p0 = g0_ref[:, 0:8]
    p1 = g1_ref[:, 0:8]
    m = 0.5 * (p0 + p1)
    q = jnp.dot(m, qwT_ref[...], preferred_element_type=_F32) \
        + cons_ref[0, :][None, :]
    ohs = []
    for hd in range(HEADS):
        qh = q[:, hd * DH:(hd + 1) * DH]
        logits = jnp.dot(qh, kT_ref[hd * DH:(hd + 1) * DH, :],
                         preferred_element_type=_F32) * 0.25
        mx = jnp.max(logits, axis=1, keepdims=True)
        ex = jnp.exp(logits - mx)
        att = ex / jnp.sum(ex, axis=1, keepdims=True)
        ohs.append(jnp.dot(att, v_ref[:, hd * DH:(hd + 1) * DH],
                           preferred_element_type=_F32))
    oh = jnp.concatenate(ohs, axis=1)
    gtok = jnp.dot(oh, woT_ref[...], preferred_element_type=_F32) \
        + cons_ref[1, :][None, :]
    gtok_ref[...] = gtok
    s = p1 - p0
    sp_ref[...] = s
    cat = jnp.concatenate(
        [s[:, 0:3], gtok, jnp.zeros((n, 128 - 67), _F32)], axis=1)
    mt = jnp.mean(cat, axis=0, keepdims=True)
    cc = cat - mt
    outer = lambda a, b: lax.dot_general(a, b, (((0,), (0,)), ((), ())),
                                         preferred_element_type=_F32)
    s2t = outer(cc, cc) + float(n) * outer(mt, mt)

    @pl.when(step == 0)
    def _():
        s1_ref[...] = jnp.zeros_like(s1_ref)
        s2_ref[...] = jnp.zeros_like(s2_ref)

    s1_ref[...] += jnp.broadcast_to(mt * float(n), s1_ref.shape)
    s2_ref[...] += s2t


def _fold_body(s1_ref, s2_ref, wpad_ref, bngT_ref, bnbT_ref, perm_ref,
               ap_ref, bp_ref):
    mu = s1_ref[0, :][None, :] / float(E)
    outer0 = lambda a, b: lax.dot_general(a, b, (((0,), (0,)), ((), ())),
                                          preferred_element_type=_F32)
    C = s2_ref[...] / float(E) - outer0(mu, mu)
    P = perm_ref[...]
    for l in range(3):
        A = wpad_ref[l]
        AC = jnp.dot(A, C, preferred_element_type=_F32)
        var = jnp.sum(AC * A, axis=1, keepdims=True)
        amu = lax.dot_general(A, mu, (((1,), (1,)), ((), ())),
                              preferred_element_type=_F32)
        scale = bngT_ref[:, l:l + 1] / jnp.sqrt(var + 1e-5)
        Ap = A * scale
        bp = bnbT_ref[:, l:l + 1] - amu * scale
        app = jnp.dot(P, Ap, preferred_element_type=_F32)
        bpp = lax.dot_general(bp, P, (((0,), (1,)), ((), ())),
                              preferred_element_type=_F32)
        ap_ref[l] = app
        bp_ref[l] = jnp.broadcast_to(bpp, (8, 256))


def _edge_body(sp_ref, gtok_ref, hg_ref, ap_ref, bp_ref,
               rt_ref, rm_ref, msg_ref):
    n = sp_ref.shape[0]
    s = sp_ref[...]
    cat = jnp.concatenate(
        [s[:, 0:3], gtok_ref[...], jnp.zeros((n, 128 - 67), _F32)], axis=1)
    z = lax.dot_general(cat, ap_ref[...], (((1,), (1,)), ((), ())),
                        preferred_element_type=_F32)
    ew = jnp.maximum(z + bp_ref[0, :][None, :], 0.0)
    hgr = jnp.dot(hg_ref[:, 8:24], rt_ref[...], preferred_element_type=_F32)
    msg = jnp.dot(ew * hgr, rm_ref[...], preferred_element_type=_F32)
    msg_ref[...] = jnp.concatenate(
        [msg, jnp.zeros((n, 112), _F32)], axis=1)


def _update_body(h_ref, aggp_ref, cntp_ref, rw_ref, cb_ref, out_ref, *, l):
    agg = aggp_ref[0][:, 0:16] + aggp_ref[1][:, 0:16]
    cnt = jnp.maximum(cntp_ref[0][:, 0:16] + cntp_ref[1][:, 0:16], 1.0)
    r = lax.dot_general(h_ref[...], rw_ref[l], (((1,), (1,)), ((), ())),
                        preferred_element_type=_F32)
    out_ref[...] = jnp.maximum(agg / cnt + r + cb_ref[l, :][None, :], 0.0)


def _sc_worker_id():
    return lax.axis_index("s") * 2 + lax.axis_index("c")


def _sc_gpair(e0_hbm, e1_hbm, ntab_hbm, g0_out, g1_out,
              idx0_v, idx1_v, g0_v, g1_v, sem0, sem1):
    wid = _sc_worker_id()
    pltpu.sync_copy(e0_hbm.at[pl.ds(wid * CPW, CPW)], idx0_v)
    pltpu.sync_copy(e1_hbm.at[pl.ds(wid * CPW, CPW)], idx1_v)

    def body(j, _):
        off = (wid * CPW + j) * CHUNK
        cp0 = pltpu.async_copy(ntab_hbm.at[idx0_v.at[j]], g0_v, sem0)
        cp1 = pltpu.async_copy(ntab_hbm.at[idx1_v.at[j]], g1_v, sem1)
        cp0.wait()
        cp1.wait()
        pltpu.sync_copy(g0_v, g0_out.at[pl.ds(off, CHUNK)])
        pltpu.sync_copy(g1_v, g1_out.at[pl.ds(off, CHUNK)])
        return 0

    lax.fori_loop(0, CPW, body, 0)


def _sc_gone(e0_hbm, ntab_hbm, g0_out, idx0_v, g0_v, sem0):
    wid = _sc_worker_id()
    pltpu.sync_copy(e0_hbm.at[pl.ds(wid * CPW, CPW)], idx0_v)

    def body(j, _):
        off = (wid * CPW + j) * CHUNK
        pltpu.async_copy(ntab_hbm.at[idx0_v.at[j]], g0_v, sem0).wait()
        pltpu.sync_copy(g0_v, g0_out.at[pl.ds(off, CHUNK)])
        return 0

    lax.fori_loop(0, CPW, body, 0)


def _sc_cnt(e1_hbm, ones_hbm, zer_hbm, cnt_out,
            idx1_v, ones_v, zer_v, cnt_sh, sem0):
    c = lax.axis_index("c")
    s = lax.axis_index("s")
    wid = _sc_worker_id()
    pltpu.sync_copy(ones_hbm, ones_v)
    pltpu.sync_copy(zer_hbm, zer_v)

    @pl.when(s < 10)
    def _():
        def zp(k, _):
            pltpu.sync_copy(zer_v, cnt_sh.at[pl.ds(s * 1000 + k * 200, 200)])
            return 0
        lax.fori_loop(0, 5, zp, 0)

    plsc.subcore_barrier()
    pltpu.sync_copy(e1_hbm.at[pl.ds(wid * CPW, CPW)], idx1_v)

    def body(j, _):
        row = wid * CPW + j

        @pl.when(row < NCHK)
        def _():
            pltpu.sync_copy(ones_v, cnt_sh.at[idx1_v.at[j]], add=True)

        return 0

    lax.fori_loop(0, CPW, body, 0)
    plsc.subcore_barrier()

    @pl.when(s < 10)
    def _():
        pltpu.sync_copy(cnt_sh.at[pl.ds(s * 1000, 1000)],
                        cnt_out.at[c, pl.ds(s * 1000, 1000)])


def _sc_scat(e1_hbm, msg_hbm, zer_hbm, agg_out, idx1_v, msg_v, zer_v,
             agg_sh, sem0):
    c = lax.axis_index("c")
    s = lax.axis_index("s")
    wid = _sc_worker_id()
    pltpu.sync_copy(zer_hbm, zer_v)
    pltpu.sync_copy(e1_hbm.at[pl.ds(wid * CPW, CPW)], idx1_v)

    @pl.when(s < 10)
    def _():
        def zp(k, _):
            pltpu.sync_copy(zer_v, agg_sh.at[pl.ds(s * 1000 + k * 200, 200)])
            return 0
        lax.fori_loop(0, 5, zp, 0)

    plsc.subcore_barrier()

    def body(j, _):
        row = wid * CPW + j

        @pl.when(row < NCHK)
        def _():
            pltpu.sync_copy(msg_hbm.at[pl.ds(row * CHUNK, CHUNK)], msg_v)
            pltpu.sync_copy(msg_v, agg_sh.at[idx1_v.at[j]], add=True)

        return 0

    lax.fori_loop(0, CPW, body, 0)
    plsc.subcore_barrier()

    @pl.when(s < 10)
    def _():
        pltpu.sync_copy(agg_sh.at[pl.ds(s * 1000, 1000)],
                        agg_out.at[c, pl.ds(s * 1000, 1000)])


def _sc_mesh():
    return plsc.VectorSubcoreMesh(core_axis_name="c", subcore_axis_name="s")


def _run_sc_gpair(e0r, e1r, ntab):
    return pl.kernel(
        _sc_gpair,
        out_type=(
            jax.ShapeDtypeStruct((EPAD, 128), _F32),
            jax.ShapeDtypeStruct((EPAD, 128), _F32),
        ),
        mesh=_sc_mesh(),
        scratch_types=[
            pltpu.VMEM((CPW, CHUNK), jnp.int32),
            pltpu.VMEM((CPW, CHUNK), jnp.int32),
            pltpu.VMEM((CHUNK, 128), _F32),
            pltpu.VMEM((CHUNK, 128), _F32),
            pltpu.SemaphoreType.DMA,
            pltpu.SemaphoreType.DMA,
        ],
    )(e0r, e1r, ntab)


def _run_sc_gone(e0r, ntab):
    return pl.kernel(
        _sc_gone,
        out_type=jax.ShapeDtypeStruct((EPAD, 128), _F32),
        mesh=_sc_mesh(),
        scratch_types=[
            pltpu.VMEM((CPW, CHUNK), jnp.int32),
            pltpu.VMEM((CHUNK, 128), _F32),
            pltpu.SemaphoreType.DMA,
        ],
    )(e0r, ntab)


def _run_sc_cnt(e1r, ones128, zer200):
    return pl.kernel(
        _sc_cnt,
        out_type=jax.ShapeDtypeStruct((2, N_S, 128), _F32),
        mesh=_sc_mesh(),
        scratch_types=[
            pltpu.VMEM((CPW, CHUNK), jnp.int32),
            pltpu.VMEM((CHUNK, 128), _F32),
            pltpu.VMEM((200, 128), _F32),
            pltpu.VMEM_SHARED((N_S, 128), _F32),
            pltpu.SemaphoreType.DMA,
        ],
    )(e1r, ones128, zer200)


def _run_sc_scat(e1r, msg128, zer200):
    return pl.kernel(
        _sc_scat,
        out_type=jax.ShapeDtypeStruct((2, N_S, 128), _F32),
        mesh=_sc_mesh(),
        scratch_types=[
            pltpu.VMEM((CPW, CHUNK), jnp.int32),
            pltpu.VMEM((CHUNK, 128), _F32),
            pltpu.VMEM((200, 128), _F32),
            pltpu.VMEM_SHARED((N_S, 128), _F32),
            pltpu.SemaphoreType.DMA,
        ],
    )(e1r, msg128, zer200)


def kernel(x, pos, batch, x_skip, pos_skip, batch_skip, edge_index,
           global_token, Wq, bq, Wk, bk, Wv, bv, in_proj_w, in_proj_b,
           out_proj_w, out_proj_b, mlp_W, mlp_b, bn_g, bn_b, root_W, conv_b):
    f32 = _F32
    posT_pad = jnp.pad(pos.T, ((0, 5), (0, NCP - N_C)),
                       constant_values=1e12)
    x_pad = jnp.pad(x, ((0, NCP - N_C), (0, 0)))
    pos_pad = jnp.pad(pos_skip, ((0, 0), (0, 5)))
    e0r = jnp.pad(edge_index[0], (0, EPAD - E)).reshape(EPAD // CHUNK, CHUNK)
    e1r = jnp.pad(edge_index[1], (0, EPAD - E)).reshape(EPAD // CHUNK, CHUNK)
    wq_pad = jnp.pad(Wq, ((0, 0), (0, 5)))
    bias = jnp.stack([bq, bk, bv, in_proj_b[:EMB], in_proj_b[EMB:2 * EMB],
                      in_proj_b[2 * EMB:], out_proj_b,
                      jnp.zeros((EMB,), f32)])
    pe = jnp.asarray(_PE)
    wpad = jnp.pad(mlp_W, ((0, 0), (0, 0), (0, 128 - 67)))
    perm = jnp.asarray(_PERM)
    rt = jnp.asarray(_RT)
    rm = jnp.asarray(_RM)

    def node_table(h_):
        return jnp.pad(jnp.concatenate([pos_pad, h_], axis=1),
                       ((0, 0), (0, 128 - 24)))

    h0 = pl.pallas_call(
        _knn_body,
        grid=(N_S // TS,),
        in_specs=[
            pl.BlockSpec((TS, 3), lambda i: (i, 0)),
            pl.BlockSpec((8, NCP), lambda i: (0, 0)),
            pl.BlockSpec((NCP, 8), lambda i: (0, 0)),
            pl.BlockSpec((TS, 8), lambda i: (i, 0)),
        ],
        out_specs=pl.BlockSpec((TS, 16), lambda i: (i, 0)),
        out_shape=jax.ShapeDtypeStruct((N_S, 16), f32),
    )(pos_skip, posT_pad, x_pad, x_skip)

    g0e, g1e = _run_sc_gpair(e0r, e1r, node_table(h0))
    ones128 = jnp.pad(jnp.ones((CHUNK, 16), f32), ((0, 0), (0, 112)))
    zer200 = jnp.zeros((200, 128), f32)
    cntp = _run_sc_cnt(e1r, ones128, zer200)

    kT, vproj, qwT, cons, woT = pl.pallas_call(
        _prologue_body,
        grid=(1,),
        in_specs=[pl.BlockSpec(s, lambda i: (0, 0))
                  for s in ((T, 1024), (EMB, 1024), (EMB, 1024), (192, EMB),
                            (EMB, 8), (EMB, EMB), (8, EMB), (T, EMB))],
        out_specs=[
            pl.BlockSpec((EMB, T), lambda i: (0, 0)),
            pl.BlockSpec((T, EMB), lambda i: (0, 0)),
            pl.BlockSpec((8, EMB), lambda i: (0, 0)),
            pl.BlockSpec((8, EMB), lambda i: (0, 0)),
            pl.BlockSpec((EMB, EMB), lambda i: (0, 0)),
        ],
        out_shape=[
            jax.ShapeDtypeStruct((EMB, T), f32),
            jax.ShapeDtypeStruct((T, EMB), f32),
            jax.ShapeDtypeStruct((8, EMB), f32),
            jax.ShapeDtypeStruct((8, EMB), f32),
            jax.ShapeDtypeStruct((EMB, EMB), f32),
        ],
    )(global_token, Wk, Wv, in_proj_w, wq_pad, out_proj_w, bias, pe)

    gtok, sp, s1, s2 = pl.pallas_call(
        _attn_body,
        grid=(E // TE,),
        in_specs=[
            pl.BlockSpec((TE, 128), lambda i: (i, 0)),
            pl.BlockSpec((TE, 128), lambda i: (i, 0)),
            pl.BlockSpec((EMB, T), lambda i: (0, 0)),
            pl.BlockSpec((T, EMB), lambda i: (0, 0)),
            pl.BlockSpec((8, EMB), lambda i: (0, 0)),
            pl.BlockSpec((8, EMB), lambda i: (0, 0)),
            pl.BlockSpec((EMB, EMB), lambda i: (0, 0)),
        ],
        out_specs=[
            pl.BlockSpec((TE, EMB), lambda i: (i, 0)),
            pl.BlockSpec((TE, 8), lambda i: (i, 0)),
            pl.BlockSpec((8, 128), lambda i: (0, 0)),
            pl.BlockSpec((128, 128), lambda i: (0, 0)),
        ],
        out_shape=[
            jax.ShapeDtypeStruct((E, EMB), f32),
            jax.ShapeDtypeStruct((E, 8), f32),
            jax.ShapeDtypeStruct((8, 128), f32),
            jax.ShapeDtypeStruct((128, 128), f32),
        ],
    )(g0e, g1e, kT, vproj, qwT, cons, woT)

    ap_all, bp_all = pl.pallas_call(
        _fold_body,
        grid=(1,),
        in_specs=[
            pl.BlockSpec((8, 128), lambda i: (0, 0)),
            pl.BlockSpec((128, 128), lambda i: (0, 0)),
            pl.BlockSpec((3, 256, 128), lambda i: (0, 0, 0)),
            pl.BlockSpec((256, 3), lambda i: (0, 0)),
            pl.BlockSpec((256, 3), lambda i: (0, 0)),
            pl.BlockSpec((256, 256), lambda i: (0, 0)),
        ],
        out_specs=[
            pl.BlockSpec((3, 256, 128), lambda i: (0, 0, 0)),
            pl.BlockSpec((3, 8, 256), lambda i: (0, 0, 0)),
        ],
        out_shape=[
            jax.ShapeDtypeStruct((3, 256, 128), f32),
            jax.ShapeDtypeStruct((3, 8, 256), f32),
        ],
    )(s1, s2, wpad, bn_g.T, bn_b.T, perm)

    h = h0
    hg = g0e
    for l in range(3):
        msg = pl.pallas_call(
            _edge_body,
            grid=(E // TE,),
            in_specs=[
                pl.BlockSpec((TE, 8), lambda i: (i, 0)),
                pl.BlockSpec((TE, EMB), lambda i: (i, 0)),
                pl.BlockSpec((TE, 128), lambda i: (i, 0)),
                pl.BlockSpec((256, 128), lambda i: (0, 0)),
                pl.BlockSpec((8, 256), lambda i: (0, 0)),
                pl.BlockSpec((16, 256), lambda i: (0, 0)),
                pl.BlockSpec((256, 16), lambda i: (0, 0)),
            ],
            out_specs=pl.BlockSpec((TE, 128), lambda i: (i, 0)),
            out_shape=jax.ShapeDtypeStruct((E, 128), f32),
        )(sp, gtok, hg, ap_all[l], bp_all[l], rt, rm)

        aggp = _run_sc_scat(e1r, msg, zer200)

        h = pl.pallas_call(
            lambda *a, l=l: _update_body(*a, l=l),
            grid=(1,),
            in_specs=[
                pl.BlockSpec((N_S, 16), lambda i: (0, 0)),
                pl.BlockSpec((2, N_S, 128), lambda i: (0, 0, 0)),
                pl.BlockSpec((2, N_S, 128), lambda i: (0, 0, 0)),
                pl.BlockSpec((3, 16, 16), lambda i: (0, 0, 0)),
                pl.BlockSpec((3, 16), lambda i: (0, 0)),
            ],
            out_specs=pl.BlockSpec((N_S, 16), lambda i: (0, 0)),
            out_shape=jax.ShapeDtypeStruct((N_S, 16), f32),
        )(h, aggp, cntp, root_W, conv_b)

        if l < 2:
            hg = _run_sc_gone(e0r, node_table(h))

    return (h, pos_skip, batch_skip)


# speedup vs baseline: 2.3167x; 1.0212x over previous
import numpy as np
import jax
import jax.numpy as jnp
from jax import lax
from jax.experimental import pallas as pl
from jax.experimental.pallas import tpu as pltpu
from jax.experimental.pallas import tpu_sc as plsc

N_C, N_S, E = 2500, 10000, 160000
EMB, HEADS, DH, T = 64, 4, 16, 64
NCP = 2560
TS = 400
TE = 4000
CHUNK = 128
NCHK = E // CHUNK
NWORK = 32
CPW = 40
CHUNKG = 128
CPWG = 40
EPAD = NWORK * CPW * CHUNK

_F32 = jnp.float32


def _make_pe(L, d):
    position = np.arange(L, dtype=np.float32)[:, None]
    div = np.exp(np.arange(0, d, 2).astype(np.float32) * (-np.log(10000.0) / d))
    pe = np.zeros((L, d), dtype=np.float32)
    pe[:, 0::2] = np.sin(position * div)
    pe[:, 1::2] = np.cos(position * div)
    return pe


_PE = _make_pe(T, EMB)
_PERM = np.zeros((256, 256), np.float32)
for _o in range(16):
    for _i in range(16):
        _PERM[_o * 16 + _i, _i * 16 + _o] = 1.0
_RT = np.zeros((16, 256), np.float32)
for _o in range(16):
    for _i in range(16):
        _RT[_i, _o * 16 + _i] = 1.0
_RM = np.zeros((256, 16), np.float32)
for _j in range(256):
    _RM[_j, _j // 16] = 1.0


def _knn_body(ps_ref, posT_ref, xpad_ref, xs_ref, out_ref):
    n = ps_ref.shape[0]
    d2 = jnp.zeros((n, NCP), _F32)
    for c in range(3):
        a = ps_ref[:, c:c + 1]
        b = posT_ref[c:c + 1, :]
        d2 = d2 + (a - b) ** 2
    lanes = lax.broadcasted_iota(jnp.int32, (n, NCP), 1)
    wmat = jnp.zeros((n, NCP), _F32)
    wsum = jnp.zeros((n, 1), _F32)
    dd = d2
    for _ in range(3):
        i_r = jnp.argmin(dd, axis=1)[:, None]
        m_r = jnp.min(dd, axis=1)[:, None]
        w_r = 1.0 / jnp.maximum(m_r, 1e-16)
        sel = lanes == i_r
        wmat = wmat + jnp.where(sel, w_r, 0.0)
        wsum = wsum + w_r
        dd = jnp.where(sel, jnp.inf, dd)
    xu = jnp.dot(wmat, xpad_ref[...], preferred_element_type=_F32) / wsum
    out_ref[...] = jnp.concatenate([xu, xs_ref[...]], axis=1)


def _prologue_body(gt_ref, wk_ref, wv_ref, ip_ref, wqp_ref, wo_ref, bias_ref,
                   pe_ref, kT_ref, v_ref, qwT_ref, cons_ref, woT_ref):
    gt = gt_ref[...]
    bq = bias_ref[0, :][None, :]
    bk = bias_ref[1, :][None, :]
    bv = bias_ref[2, :][None, :]
    biq = bias_ref[3, :][None, :]
    bik = bias_ref[4, :][None, :]
    biv = bias_ref[5, :][None, :]
    ob = bias_ref[6, :][None, :]
    pe = pe_ref[...]
    ct = lambda a, b: lax.dot_general(a, b, (((1,), (1,)), ((), ())),
                                      preferred_element_type=_F32)
    K = ct(gt, wk_ref[...]) + bk + pe
    V = ct(gt, wv_ref[...]) + bv + pe
    wiq = ip_ref[0:EMB]
    wik = ip_ref[EMB:2 * EMB]
    wiv = ip_ref[2 * EMB:3 * EMB]
    kproj = ct(K, wik) + bik
    vproj = ct(V, wiv) + biv
    kT_ref[...] = kproj.T
    v_ref[...] = vproj
    qw = jnp.dot(wiq, wqp_ref[...], preferred_element_type=_F32)
    qwT_ref[...] = qw.T
    qb = ct(bq, wiq) + biq
    cons = jnp.concatenate(
        [qb, ob, jnp.zeros((6, EMB), _F32)], axis=0)
    cons_ref[...] = cons
    woT_ref[...] = wo_ref[...].T


def _attn_body(g0_ref, g1_ref, kT_ref, v_ref, qwT_ref, cons_ref, woT_ref,
               gtok_ref, sp_ref, s1_ref, s2_ref):
    step = pl.program_id(0)
    n = g0_ref.shape[0]
    p0 = g0_ref[:, 0:8]
    p1 = g1_ref[:, 0:8]
    m = 0.5 * (p0 + p1)
    q = jnp.dot(m, qwT_ref[...], preferred_element_type=_F32) \
        + cons_ref[0, :][None, :]
    ohs = []
    for hd in range(HEADS):
        qh = q[:, hd * DH:(hd + 1) * DH]
        logits = jnp.dot(qh, kT_ref[hd * DH:(hd + 1) * DH, :],
                         preferred_element_type=_F32) * 0.25
        mx = jnp.max(logits, axis=1, keepdims=True)
        ex = jnp.exp(logits - mx)
        att = ex / jnp.sum(ex, axis=1, keepdims=True)
        ohs.append(jnp.dot(att, v_ref[:, hd * DH:(hd + 1) * DH],
                           preferred_element_type=_F32))
    oh = jnp.concatenate(ohs, axis=1)
    gtok = jnp.dot(oh, woT_ref[...], preferred_element_type=_F32) \
        + cons_ref[1, :][None, :]
    gtok_ref[...] = gtok
    s = p1 - p0
    sp_ref[...] = s
    cat = jnp.concatenate(
        [s[:, 0:3], gtok, jnp.zeros((n, 128 - 67), _F32)], axis=1)
    mt = jnp.mean(cat, axis=0, keepdims=True)
    cc = cat - mt
    outer = lambda a, b: lax.dot_general(a, b, (((0,), (0,)), ((), ())),
                                         preferred_element_type=_F32)
    s2t = outer(cc, cc) + float(n) * outer(mt, mt)

    @pl.when(step == 0)
    def _():
        s1_ref[...] = jnp.zeros_like(s1_ref)
        s2_ref[...] = jnp.zeros_like(s2_ref)

    s1_ref[...] += jnp.broadcast_to(mt * float(n), s1_ref.shape)
    s2_ref[...] += s2t


def _fold_body(s1_ref, s2_ref, wpad_ref, bngT_ref, bnbT_ref, perm_ref,
               ap_ref, bp_ref):
    mu = s1_ref[0, :][None, :] / float(E)
    outer0 = lambda a, b: lax.dot_general(a, b, (((0,), (0,)), ((), ())),
                                          preferred_element_type=_F32)
    C = s2_ref[...] / float(E) - outer0(mu, mu)
    P = perm_ref[...]
    for l in range(3):
        A = wpad_ref[l]
        AC = jnp.dot(A, C, preferred_element_type=_F32)
        var = jnp.sum(AC * A, axis=1, keepdims=True)
        amu = lax.dot_general(A, mu, (((1,), (1,)), ((), ())),
                              preferred_element_type=_F32)
        scale = bngT_ref[:, l:l + 1] / jnp.sqrt(var + 1e-5)
        Ap = A * scale
        bp = bnbT_ref[:, l:l + 1] - amu * scale
        app = jnp.dot(P, Ap, preferred_element_type=_F32)
        bpp = lax.dot_general(bp, P, (((0,), (1,)), ((), ())),
                              preferred_element_type=_F32)
        ap_ref[l] = app
        bp_ref[l] = jnp.broadcast_to(bpp, (8, 256))


def _edge_body(sp_ref, gtok_ref, hg_ref, ap_ref, bp_ref,
               rt_ref, rm_ref, msg_ref):
    n = sp_ref.shape[0]
    s = sp_ref[...]
    cat = jnp.concatenate(
        [s[:, 0:3], gtok_ref[...], jnp.zeros((n, 128 - 67), _F32)], axis=1)
    z = lax.dot_general(cat, ap_ref[...], (((1,), (1,)), ((), ())),
                        preferred_element_type=_F32)
    ew = jnp.maximum(z + bp_ref[0, :][None, :], 0.0)
    hgr = jnp.dot(hg_ref[:, 8:24], rt_ref[...], preferred_element_type=_F32)
    msg = jnp.dot(ew * hgr, rm_ref[...], preferred_element_type=_F32)
    msg_ref[...] = jnp.concatenate(
        [msg, jnp.zeros((n, 112), _F32)], axis=1)


def _update_body(h_ref, aggp_ref, cntp_ref, rw_ref, cb_ref, out_ref, *, l):
    agg = aggp_ref[0][:, 0:16] + aggp_ref[1][:, 0:16]
    cnt = jnp.maximum(cntp_ref[0][:, 0:16] + cntp_ref[1][:, 0:16], 1.0)
    r = lax.dot_general(h_ref[...], rw_ref[l], (((1,), (1,)), ((), ())),
                        preferred_element_type=_F32)
    out_ref[...] = jnp.maximum(agg / cnt + r + cb_ref[l, :][None, :], 0.0)


def _sc_worker_id():
    return lax.axis_index("s") * 2 + lax.axis_index("c")


def _sc_gpair(e0_hbm, e1_hbm, ntab_hbm, g0_out, g1_out,
              idx0_v, idx1_v, g0a, g1a, g0b, g1b,
              s0a, s1a, s0b, s1b):
    wid = _sc_worker_id()
    pltpu.sync_copy(e0_hbm.at[pl.ds(wid * CPWG, CPWG)], idx0_v)
    pltpu.sync_copy(e1_hbm.at[pl.ds(wid * CPWG, CPWG)], idx1_v)

    def body(k, _):
        j0 = 2 * k
        j1 = 2 * k + 1
        off0 = (wid * CPWG + j0) * CHUNKG
        off1 = (wid * CPWG + j1) * CHUNKG
        ca0 = pltpu.async_copy(ntab_hbm.at[idx0_v.at[j0]], g0a, s0a)
        ca1 = pltpu.async_copy(ntab_hbm.at[idx1_v.at[j0]], g1a, s1a)
        cb0 = pltpu.async_copy(ntab_hbm.at[idx0_v.at[j1]], g0b, s0b)
        cb1 = pltpu.async_copy(ntab_hbm.at[idx1_v.at[j1]], g1b, s1b)
        ca0.wait()
        ca1.wait()
        pltpu.sync_copy(g0a, g0_out.at[pl.ds(off0, CHUNKG)])
        pltpu.sync_copy(g1a, g1_out.at[pl.ds(off0, CHUNKG)])
        cb0.wait()
        cb1.wait()
        pltpu.sync_copy(g0b, g0_out.at[pl.ds(off1, CHUNKG)])
        pltpu.sync_copy(g1b, g1_out.at[pl.ds(off1, CHUNKG)])
        return 0

    lax.fori_loop(0, CPWG // 2, body, 0)


def _sc_gone(e0_hbm, ntab_hbm, g0_out, idx0_v, g0a, g0b, s0a, s0b):
    wid = _sc_worker_id()
    pltpu.sync_copy(e0_hbm.at[pl.ds(wid * CPWG, CPWG)], idx0_v)

    def body(k, _):
        j0 = 2 * k
        j1 = 2 * k + 1
        off0 = (wid * CPWG + j0) * CHUNKG
        off1 = (wid * CPWG + j1) * CHUNKG
        ca = pltpu.async_copy(ntab_hbm.at[idx0_v.at[j0]], g0a, s0a)
        cb = pltpu.async_copy(ntab_hbm.at[idx0_v.at[j1]], g0b, s0b)
        ca.wait()
        pltpu.sync_copy(g0a, g0_out.at[pl.ds(off0, CHUNKG)])
        cb.wait()
        pltpu.sync_copy(g0b, g0_out.at[pl.ds(off1, CHUNKG)])
        return 0

    lax.fori_loop(0, CPWG // 2, body, 0)


def _sc_cnt(e1_hbm, ones_hbm, zer_hbm, cnt_out,
            idx1_v, ones_v, zer_v, cnt_sh, sem0):
    c = lax.axis_index("c")
    s = lax.axis_index("s")
    wid = _sc_worker_id()
    pltpu.sync_copy(ones_hbm, ones_v)
    pltpu.sync_copy(zer_hbm, zer_v)

    @pl.when(s < 10)
    def _():
        def zp(k, _):
            pltpu.sync_copy(zer_v, cnt_sh.at[pl.ds(s * 1000 + k * 200, 200)])
            return 0
        lax.fori_loop(0, 5, zp, 0)

    plsc.subcore_barrier()
    pltpu.sync_copy(e1_hbm.at[pl.ds(wid * CPW, CPW)], idx1_v)

    def body(j, _):
        row = wid * CPW + j

        @pl.when(row < NCHK)
        def _():
            pltpu.sync_copy(ones_v, cnt_sh.at[idx1_v.at[j]], add=True)

        return 0

    lax.fori_loop(0, CPW, body, 0)
    plsc.subcore_barrier()

    @pl.when(s < 10)
    def _():
        pltpu.sync_copy(cnt_sh.at[pl.ds(s * 1000, 1000)],
                        cnt_out.at[c, pl.ds(s * 1000, 1000)])


def _sc_scat(e1_hbm, msg_hbm, zer_hbm, agg_out, idx1_v, msg_v, zer_v,
             agg_sh, semm):
    c = lax.axis_index("c")
    s = lax.axis_index("s")
    wid = _sc_worker_id()
    pltpu.sync_copy(zer_hbm, zer_v)
    pltpu.sync_copy(e1_hbm.at[pl.ds(wid * CPW, CPW)], idx1_v)

    @pl.when(s < 10)
    def _():
        def zp(k, _):
            pltpu.sync_copy(zer_v, agg_sh.at[pl.ds(s * 1000 + k * 200, 200)])
            return 0
        lax.fori_loop(0, 5, zp, 0)

    plsc.subcore_barrier()

    def body(j, _):
        row = wid * CPW + j

        @pl.when(row < NCHK)
        def _():
            pltpu.sync_copy(msg_hbm.at[pl.ds(row * CHUNK, CHUNK)], msg_v)
            pltpu.sync_copy(msg_v, agg_sh.at[idx1_v.at[j]], add=True)

        return 0

    lax.fori_loop(0, CPW, body, 0)
    plsc.subcore_barrier()

    @pl.when(s < 10)
    def _():
        pltpu.sync_copy(agg_sh.at[pl.ds(s * 1000, 1000)],
                        agg_out.at[c, pl.ds(s * 1000, 1000)])


def _sc_mesh():
    return plsc.VectorSubcoreMesh(core_axis_name="c", subcore_axis_name="s")


def _run_sc_gpair(e0g, e1g, ntab):
    return pl.kernel(
        _sc_gpair,
        out_type=(
            jax.ShapeDtypeStruct((EPAD, 128), _F32),
            jax.ShapeDtypeStruct((EPAD, 128), _F32),
        ),
        mesh=_sc_mesh(),
        scratch_types=[
            pltpu.VMEM((CPWG, CHUNKG), jnp.int32),
            pltpu.VMEM((CPWG, CHUNKG), jnp.int32),
            pltpu.VMEM((CHUNKG, 128), _F32),
            pltpu.VMEM((CHUNKG, 128), _F32),
            pltpu.VMEM((CHUNKG, 128), _F32),
            pltpu.VMEM((CHUNKG, 128), _F32),
            pltpu.SemaphoreType.DMA,
            pltpu.SemaphoreType.DMA,
            pltpu.SemaphoreType.DMA,
            pltpu.SemaphoreType.DMA,
        ],
    )(e0g, e1g, ntab)


def _run_sc_gone(e0g, ntab):
    return pl.kernel(
        _sc_gone,
        out_type=jax.ShapeDtypeStruct((EPAD, 128), _F32),
        mesh=_sc_mesh(),
        scratch_types=[
            pltpu.VMEM((CPWG, CHUNKG), jnp.int32),
            pltpu.VMEM((CHUNKG, 128), _F32),
            pltpu.VMEM((CHUNKG, 128), _F32),
            pltpu.SemaphoreType.DMA,
            pltpu.SemaphoreType.DMA,
        ],
    )(e0g, ntab)


def _run_sc_cnt(e1r, ones128, zer200):
    return pl.kernel(
        _sc_cnt,
        out_type=jax.ShapeDtypeStruct((2, N_S, 128), _F32),
        mesh=_sc_mesh(),
        scratch_types=[
            pltpu.VMEM((CPW, CHUNK), jnp.int32),
            pltpu.VMEM((CHUNK, 128), _F32),
            pltpu.VMEM((200, 128), _F32),
            pltpu.VMEM_SHARED((N_S, 128), _F32),
            pltpu.SemaphoreType.DMA,
        ],
    )(e1r, ones128, zer200)


def _run_sc_scat(e1r, msg128, zer200):
    return pl.kernel(
        _sc_scat,
        out_type=jax.ShapeDtypeStruct((2, N_S, 128), _F32),
        mesh=_sc_mesh(),
        scratch_types=[
            pltpu.VMEM((CPW, CHUNK), jnp.int32),
            pltpu.VMEM((CHUNK, 128), _F32),
            pltpu.VMEM((200, 128), _F32),
            pltpu.VMEM_SHARED((N_S, 128), _F32),
            pltpu.SemaphoreType.DMA,
        ],
    )(e1r, msg128, zer200)


def kernel(x, pos, batch, x_skip, pos_skip, batch_skip, edge_index,
           global_token, Wq, bq, Wk, bk, Wv, bv, in_proj_w, in_proj_b,
           out_proj_w, out_proj_b, mlp_W, mlp_b, bn_g, bn_b, root_W, conv_b):
    f32 = _F32
    posT_pad = jnp.pad(pos.T, ((0, 5), (0, NCP - N_C)),
                       constant_values=1e12)
    x_pad = jnp.pad(x, ((0, NCP - N_C), (0, 0)))
    pos_pad = jnp.pad(pos_skip, ((0, 0), (0, 5)))
    e0p = jnp.pad(edge_index[0], (0, EPAD - E))
    e1p = jnp.pad(edge_index[1], (0, EPAD - E))
    e0r = e0p.reshape(EPAD // CHUNK, CHUNK)
    e1r = e1p.reshape(EPAD // CHUNK, CHUNK)
    e0g = e0p.reshape(EPAD // CHUNKG, CHUNKG)
    e1g = e1p.reshape(EPAD // CHUNKG, CHUNKG)
    wq_pad = jnp.pad(Wq, ((0, 0), (0, 5)))
    bias = jnp.stack([bq, bk, bv, in_proj_b[:EMB], in_proj_b[EMB:2 * EMB],
                      in_proj_b[2 * EMB:], out_proj_b,
                      jnp.zeros((EMB,), f32)])
    pe = jnp.asarray(_PE)
    wpad = jnp.pad(mlp_W, ((0, 0), (0, 0), (0, 128 - 67)))
    perm = jnp.asarray(_PERM)
    rt = jnp.asarray(_RT)
    rm = jnp.asarray(_RM)

    def node_table(h_):
        return jnp.pad(jnp.concatenate([pos_pad, h_], axis=1),
                       ((0, 0), (0, 128 - 24)))

    h0 = pl.pallas_call(
        _knn_body,
        grid=(N_S // TS,),
        in_specs=[
            pl.BlockSpec((TS, 3), lambda i: (i, 0)),
            pl.BlockSpec((8, NCP), lambda i: (0, 0)),
            pl.BlockSpec((NCP, 8), lambda i: (0, 0)),
            pl.BlockSpec((TS, 8), lambda i: (i, 0)),
        ],
        out_specs=pl.BlockSpec((TS, 16), lambda i: (i, 0)),
        out_shape=jax.ShapeDtypeStruct((N_S, 16), f32),
    )(pos_skip, posT_pad, x_pad, x_skip)

    g0e, g1e = _run_sc_gpair(e0g, e1g, node_table(h0))
    ones128 = jnp.pad(jnp.ones((CHUNK, 16), f32), ((0, 0), (0, 112)))
    zer200 = jnp.zeros((200, 128), f32)
    cntp = _run_sc_cnt(e1r, ones128, zer200)

    kT, vproj, qwT, cons, woT = pl.pallas_call(
        _prologue_body,
        grid=(1,),
        in_specs=[pl.BlockSpec(s, lambda i: (0, 0))
                  for s in ((T, 1024), (EMB, 1024), (EMB, 1024), (192, EMB),
                            (EMB, 8), (EMB, EMB), (8, EMB), (T, EMB))],
        out_specs=[
            pl.BlockSpec((EMB, T), lambda i: (0, 0)),
            pl.BlockSpec((T, EMB), lambda i: (0, 0)),
            pl.BlockSpec((8, EMB), lambda i: (0, 0)),
            pl.BlockSpec((8, EMB), lambda i: (0, 0)),
            pl.BlockSpec((EMB, EMB), lambda i: (0, 0)),
        ],
        out_shape=[
            jax.ShapeDtypeStruct((EMB, T), f32),
            jax.ShapeDtypeStruct((T, EMB), f32),
            jax.ShapeDtypeStruct((8, EMB), f32),
            jax.ShapeDtypeStruct((8, EMB), f32),
            jax.ShapeDtypeStruct((EMB, EMB), f32),
        ],
    )(global_token, Wk, Wv, in_proj_w, wq_pad, out_proj_w, bias, pe)

    gtok, sp, s1, s2 = pl.pallas_call(
        _attn_body,
        grid=(E // TE,),
        in_specs=[
            pl.BlockSpec((TE, 128), lambda i: (i, 0)),
            pl.BlockSpec((TE, 128), lambda i: (i, 0)),
            pl.BlockSpec((EMB, T), lambda i: (0, 0)),
            pl.BlockSpec((T, EMB), lambda i: (0, 0)),
            pl.BlockSpec((8, EMB), lambda i: (0, 0)),
            pl.BlockSpec((8, EMB), lambda i: (0, 0)),
            pl.BlockSpec((EMB, EMB), lambda i: (0, 0)),
        ],
        out_specs=[
            pl.BlockSpec((TE, EMB), lambda i: (i, 0)),
            pl.BlockSpec((TE, 8), lambda i: (i, 0)),
            pl.BlockSpec((8, 128), lambda i: (0, 0)),
            pl.BlockSpec((128, 128), lambda i: (0, 0)),
        ],
        out_shape=[
            jax.ShapeDtypeStruct((E, EMB), f32),
            jax.ShapeDtypeStruct((E, 8), f32),
            jax.ShapeDtypeStruct((8, 128), f32),
            jax.ShapeDtypeStruct((128, 128), f32),
        ],
    )(g0e, g1e, kT, vproj, qwT, cons, woT)

    ap_all, bp_all = pl.pallas_call(
        _fold_body,
        grid=(1,),
        in_specs=[
            pl.BlockSpec((8, 128), lambda i: (0, 0)),
            pl.BlockSpec((128, 128), lambda i: (0, 0)),
            pl.BlockSpec((3, 256, 128), lambda i: (0, 0, 0)),
            pl.BlockSpec((256, 3), lambda i: (0, 0)),
            pl.BlockSpec((256, 3), lambda i: (0, 0)),
            pl.BlockSpec((256, 256), lambda i: (0, 0)),
        ],
        out_specs=[
            pl.BlockSpec((3, 256, 128), lambda i: (0, 0, 0)),
            pl.BlockSpec((3, 8, 256), lambda i: (0, 0, 0)),
        ],
        out_shape=[
            jax.ShapeDtypeStruct((3, 256, 128), f32),
            jax.ShapeDtypeStruct((3, 8, 256), f32),
        ],
    )(s1, s2, wpad, bn_g.T, bn_b.T, perm)

    h = h0
    hg = g0e
    for l in range(3):
        msg = pl.pallas_call(
            _edge_body,
            grid=(E // TE,),
            in_specs=[
                pl.BlockSpec((TE, 8), lambda i: (i, 0)),
                pl.BlockSpec((TE, EMB), lambda i: (i, 0)),
                pl.BlockSpec((TE, 128), lambda i: (i, 0)),
                pl.BlockSpec((256, 128), lambda i: (0, 0)),
                pl.BlockSpec((8, 256), lambda i: (0, 0)),
                pl.BlockSpec((16, 256), lambda i: (0, 0)),
                pl.BlockSpec((256, 16), lambda i: (0, 0)),
            ],
            out_specs=pl.BlockSpec((TE, 128), lambda i: (i, 0)),
            out_shape=jax.ShapeDtypeStruct((E, 128), f32),
        )(sp, gtok, hg, ap_all[l], bp_all[l], rt, rm)

        aggp = _run_sc_scat(e1r, msg, zer200)

        h = pl.pallas_call(
            lambda *a, l=l: _update_body(*a, l=l),
            grid=(1,),
            in_specs=[
                pl.BlockSpec((N_S, 16), lambda i: (0, 0)),
                pl.BlockSpec((2, N_S, 128), lambda i: (0, 0, 0)),
                pl.BlockSpec((2, N_S, 128), lambda i: (0, 0, 0)),
                pl.BlockSpec((3, 16, 16), lambda i: (0, 0, 0)),
                pl.BlockSpec((3, 16), lambda i: (0, 0)),
            ],
            out_specs=pl.BlockSpec((N_S, 16), lambda i: (0, 0)),
            out_shape=jax.ShapeDtypeStruct((N_S, 16), f32),
        )(h, aggp, cntp, root_W, conv_b)

        if l < 2:
            hg = _run_sc_gone(e0g, node_table(h))

    return (h, pos_skip, batch_skip)


# speedup vs baseline: 3.6041x; 1.5557x over previous
import numpy as np
import jax
import jax.numpy as jnp
from jax import lax
from jax.experimental import pallas as pl
from jax.experimental.pallas import tpu as pltpu
from jax.experimental.pallas import tpu_sc as plsc

N_C, N_S, E = 2500, 10000, 160000
EMB, HEADS, DH, T = 64, 4, 16, 64
NCP = 2560
TS = 400
TE = 4000
CHUNK = 128
NCHK = E // CHUNK
NWORK = 32
CPW = 40
CHUNKG = 128
CPWG = 40
EPAD = NWORK * CPW * CHUNK

_F32 = jnp.float32


def _make_pe(L, d):
    position = np.arange(L, dtype=np.float32)[:, None]
    div = np.exp(np.arange(0, d, 2).astype(np.float32) * (-np.log(10000.0) / d))
    pe = np.zeros((L, d), dtype=np.float32)
    pe[:, 0::2] = np.sin(position * div)
    pe[:, 1::2] = np.cos(position * div)
    return pe


_PE = _make_pe(T, EMB)
_PERM = np.zeros((256, 256), np.float32)
for _o in range(16):
    for _i in range(16):
        _PERM[_o * 16 + _i, _i * 16 + _o] = 1.0
_RT = np.zeros((16, 256), np.float32)
for _o in range(16):
    for _i in range(16):
        _RT[_i, _o * 16 + _i] = 1.0
_RM = np.zeros((256, 16), np.float32)
for _j in range(256):
    _RM[_j, _j // 16] = 1.0


def _knn_body(ps_ref, posT_ref, xpad_ref, xs_ref, out_ref):
    n = ps_ref.shape[0]
    d2 = jnp.zeros((n, NCP), _F32)
    for c in range(3):
        a = ps_ref[:, c:c + 1]
        b = posT_ref[c:c + 1, :]
        d2 = d2 + (a - b) ** 2
    lanes = lax.broadcasted_iota(jnp.int32, (n, NCP), 1)
    wmat = jnp.zeros((n, NCP), _F32)
    wsum = jnp.zeros((n, 1), _F32)
    dd = d2
    for _ in range(3):
        i_r = jnp.argmin(dd, axis=1)[:, None]
        m_r = jnp.min(dd, axis=1)[:, None]
        w_r = 1.0 / jnp.maximum(m_r, 1e-16)
        sel = lanes == i_r
        wmat = wmat + jnp.where(sel, w_r, 0.0)
        wsum = wsum + w_r
        dd = jnp.where(sel, jnp.inf, dd)
    xu = jnp.dot(wmat, xpad_ref[...], preferred_element_type=_F32) / wsum
    out_ref[...] = jnp.concatenate([xu, xs_ref[...]], axis=1)


def _prologue_body(gt_ref, wk_ref, wv_ref, ip_ref, wqp_ref, wo_ref, bias_ref,
                   pe_ref, kT_ref, v_ref, qwT_ref, cons_ref, woT_ref):
    gt = gt_ref[...]
    bq = bias_ref[0, :][None, :]
    bk = bias_ref[1, :][None, :]
    bv = bias_ref[2, :][None, :]
    biq = bias_ref[3, :][None, :]
    bik = bias_ref[4, :][None, :]
    biv = bias_ref[5, :][None, :]
    ob = bias_ref[6, :][None, :]
    pe = pe_ref[...]
    ct = lambda a, b: lax.dot_general(a, b, (((1,), (1,)), ((), ())),
                                      preferred_element_type=_F32)
    K = ct(gt, wk_ref[...]) + bk + pe
    V = ct(gt, wv_ref[...]) + bv + pe
    wiq = ip_ref[0:EMB]
    wik = ip_ref[EMB:2 * EMB]
    wiv = ip_ref[2 * EMB:3 * EMB]
    kproj = ct(K, wik) + bik
    vproj = ct(V, wiv) + biv
    kT_ref[...] = kproj.T
    v_ref[...] = vproj
    qw = jnp.dot(wiq, wqp_ref[...], preferred_element_type=_F32)
    qwT_ref[...] = qw.T
    qb = ct(bq, wiq) + biq
    cons = jnp.concatenate(
        [qb, ob, jnp.zeros((6, EMB), _F32)], axis=0)
    cons_ref[...] = cons
    woT_ref[...] = wo_ref[...].T


def _attn_body(g0_ref, g1_ref, kT_ref, v_ref, qwT_ref, cons_ref, woT_ref,
               gtok_ref, sp_ref, s1_ref, s2_ref):
    step = pl.program_id(0)
    n = g0_ref.shape[0]
    p0 = g0_ref[:, 0:8]
    p1 = g1_ref[:, 0:8]
    m = 0.5 * (p0 + p1)
    q = jnp.dot(m, qwT_ref[...], preferred_element_type=_F32) \
        + cons_ref[0, :][None, :]
    ohs = []
    for hd in range(HEADS):
        qh = q[:, hd * DH:(hd + 1) * DH]
        logits = jnp.dot(qh, kT_ref[hd * DH:(hd + 1) * DH, :],
                         preferred_element_type=_F32) * 0.25
        mx = jnp.max(logits, axis=1, keepdims=True)
        ex = jnp.exp(logits - mx)
        att = ex / jnp.sum(ex, axis=1, keepdims=True)
        ohs.append(jnp.dot(att, v_ref[:, hd * DH:(hd + 1) * DH],
                           preferred_element_type=_F32))
    oh = jnp.concatenate(ohs, axis=1)
    gtok = jnp.dot(oh, woT_ref[...], preferred_element_type=_F32) \
        + cons_ref[1, :][None, :]
    gtok_ref[...] = gtok
    s = p1 - p0
    sp_ref[...] = s
    cat = jnp.concatenate(
        [s[:, 0:3], gtok, jnp.zeros((n, 128 - 67), _F32)], axis=1)
    mt = jnp.mean(cat, axis=0, keepdims=True)
    cc = cat - mt
    outer = lambda a, b: lax.dot_general(a, b, (((0,), (0,)), ((), ())),
                                         preferred_element_type=_F32)
    s2t = outer(cc, cc) + float(n) * outer(mt, mt)

    @pl.when(step == 0)
    def _():
        s1_ref[...] = jnp.zeros_like(s1_ref)
        s2_ref[...] = jnp.zeros_like(s2_ref)

    s1_ref[...] += jnp.broadcast_to(mt * float(n), s1_ref.shape)
    s2_ref[...] += s2t


def _fold_body(s1_ref, s2_ref, wpad_ref, bngT_ref, bnbT_ref, perm_ref,
               ap_ref, bp_ref):
    mu = s1_ref[0, :][None, :] / float(E)
    outer0 = lambda a, b: lax.dot_general(a, b, (((0,), (0,)), ((), ())),
                                          preferred_element_type=_F32)
    C = s2_ref[...] / float(E) - outer0(mu, mu)
    P = perm_ref[...]
    for l in range(3):
        A = wpad_ref[l]
        AC = jnp.dot(A, C, preferred_element_type=_F32)
        var = jnp.sum(AC * A, axis=1, keepdims=True)
        amu = lax.dot_general(A, mu, (((1,), (1,)), ((), ())),
                              preferred_element_type=_F32)
        scale = bngT_ref[:, l:l + 1] / jnp.sqrt(var + 1e-5)
        Ap = A * scale
        bp = bnbT_ref[:, l:l + 1] - amu * scale
        app = jnp.dot(P, Ap, preferred_element_type=_F32)
        bpp = lax.dot_general(bp, P, (((0,), (1,)), ((), ())),
                              preferred_element_type=_F32)
        ap_ref[l] = app
        bp_ref[l] = jnp.broadcast_to(bpp, (8, 256))


def _edge_body(sp_ref, gtok_ref, hg_ref, ap_ref, bp_ref,
               rt_ref, rm_ref, msg_ref):
    n = sp_ref.shape[0]
    s = sp_ref[...]
    cat = jnp.concatenate(
        [s[:, 0:3], gtok_ref[...], jnp.zeros((n, 128 - 67), _F32)], axis=1)
    z = lax.dot_general(cat, ap_ref[...], (((1,), (1,)), ((), ())),
                        preferred_element_type=_F32)
    ew = jnp.maximum(z + bp_ref[0, :][None, :], 0.0)
    hgr = jnp.dot(hg_ref[:, 8:24], rt_ref[...], preferred_element_type=_F32)
    msg = jnp.dot(ew * hgr, rm_ref[...], preferred_element_type=_F32)
    msg_ref[...] = jnp.concatenate(
        [msg, jnp.zeros((n, 112), _F32)], axis=1)


def _update_body(h_ref, aggp_ref, cntp_ref, rw_ref, cb_ref, out_ref, *, l):
    agg = aggp_ref[0][:, 0:16] + aggp_ref[1][:, 0:16]
    cnt = jnp.maximum(cntp_ref[0][:, 0:16] + cntp_ref[1][:, 0:16], 1.0)
    r = lax.dot_general(h_ref[...], rw_ref[l], (((1,), (1,)), ((), ())),
                        preferred_element_type=_F32)
    out_ref[...] = jnp.maximum(agg / cnt + r + cb_ref[l, :][None, :], 0.0)


def _sc_worker_id():
    return lax.axis_index("s") * 2 + lax.axis_index("c")


def _sc_gone(e0_hbm, ntab_hbm, g0_out, idx0_v, g0a, g0b, ntab_sh, s0a, s0b):
    s = lax.axis_index("s")
    wid = _sc_worker_id()

    @pl.when(s < 10)
    def _():
        pltpu.sync_copy(ntab_hbm.at[pl.ds(s * 1000, 1000)],
                        ntab_sh.at[pl.ds(s * 1000, 1000)])

    pltpu.sync_copy(e0_hbm.at[pl.ds(wid * CPWG, CPWG)], idx0_v)
    plsc.subcore_barrier()

    def body(k, _):
        j0 = 2 * k
        j1 = 2 * k + 1
        off0 = (wid * CPWG + j0) * CHUNKG
        off1 = (wid * CPWG + j1) * CHUNKG
        ca = pltpu.async_copy(ntab_sh.at[idx0_v.at[j0]], g0a, s0a)
        cb = pltpu.async_copy(ntab_sh.at[idx0_v.at[j1]], g0b, s0b)
        ca.wait()
        pltpu.sync_copy(g0a, g0_out.at[pl.ds(off0, CHUNKG)])
        cb.wait()
        pltpu.sync_copy(g0b, g0_out.at[pl.ds(off1, CHUNKG)])
        return 0

    lax.fori_loop(0, CPWG // 2, body, 0)


def _sc_cnt(e1_hbm, ones_hbm, zer_hbm, cnt_out,
            idx1_v, ones_v, zer_v, cnt_sh, sem0):
    c = lax.axis_index("c")
    s = lax.axis_index("s")
    wid = _sc_worker_id()
    pltpu.sync_copy(ones_hbm, ones_v)
    pltpu.sync_copy(zer_hbm, zer_v)

    @pl.when(s < 10)
    def _():
        def zp(k, _):
            pltpu.sync_copy(zer_v, cnt_sh.at[pl.ds(s * 1000 + k * 200, 200)])
            return 0
        lax.fori_loop(0, 5, zp, 0)

    plsc.subcore_barrier()
    pltpu.sync_copy(e1_hbm.at[pl.ds(wid * CPW, CPW)], idx1_v)

    def body(j, _):
        row = wid * CPW + j

        @pl.when(row < NCHK)
        def _():
            pltpu.sync_copy(ones_v, cnt_sh.at[idx1_v.at[j]], add=True)

        return 0

    lax.fori_loop(0, CPW, body, 0)
    plsc.subcore_barrier()

    @pl.when(s < 10)
    def _():
        pltpu.sync_copy(cnt_sh.at[pl.ds(s * 1000, 1000)],
                        cnt_out.at[c, pl.ds(s * 1000, 1000)])


def _sc_scat(e1_hbm, msg_hbm, zer_hbm, agg_out, idx1_v, msg_v, zer_v,
             agg_sh, semm):
    c = lax.axis_index("c")
    s = lax.axis_index("s")
    wid = _sc_worker_id()
    pltpu.sync_copy(zer_hbm, zer_v)
    pltpu.sync_copy(e1_hbm.at[pl.ds(wid * CPW, CPW)], idx1_v)

    @pl.when(s < 10)
    def _():
        def zp(k, _):
            pltpu.sync_copy(zer_v, agg_sh.at[pl.ds(s * 1000 + k * 200, 200)])
            return 0
        lax.fori_loop(0, 5, zp, 0)

    plsc.subcore_barrier()

    def body(j, _):
        row = wid * CPW + j

        @pl.when(row < NCHK)
        def _():
            pltpu.sync_copy(msg_hbm.at[pl.ds(row * CHUNK, CHUNK)], msg_v)
            pltpu.sync_copy(msg_v, agg_sh.at[idx1_v.at[j]], add=True)

        return 0

    lax.fori_loop(0, CPW, body, 0)
    plsc.subcore_barrier()

    @pl.when(s < 10)
    def _():
        pltpu.sync_copy(agg_sh.at[pl.ds(s * 1000, 1000)],
                        agg_out.at[c, pl.ds(s * 1000, 1000)])


def _sc_mesh():
    return plsc.VectorSubcoreMesh(core_axis_name="c", subcore_axis_name="s")


def _run_sc_gone(e0g, ntab):
    return pl.kernel(
        _sc_gone,
        out_type=jax.ShapeDtypeStruct((EPAD, 128), _F32),
        mesh=_sc_mesh(),
        scratch_types=[
            pltpu.VMEM((CPWG, CHUNKG), jnp.int32),
            pltpu.VMEM((CHUNKG, 128), _F32),
            pltpu.VMEM((CHUNKG, 128), _F32),
            pltpu.VMEM_SHARED((N_S, 128), _F32),
            pltpu.SemaphoreType.DMA,
            pltpu.SemaphoreType.DMA,
        ],
    )(e0g, ntab)


def _run_sc_cnt(e1r, ones128, zer200):
    return pl.kernel(
        _sc_cnt,
        out_type=jax.ShapeDtypeStruct((2, N_S, 128), _F32),
        mesh=_sc_mesh(),
        scratch_types=[
            pltpu.VMEM((CPW, CHUNK), jnp.int32),
            pltpu.VMEM((CHUNK, 128), _F32),
            pltpu.VMEM((200, 128), _F32),
            pltpu.VMEM_SHARED((N_S, 128), _F32),
            pltpu.SemaphoreType.DMA,
        ],
    )(e1r, ones128, zer200)


def _run_sc_scat(e1r, msg128, zer200):
    return pl.kernel(
        _sc_scat,
        out_type=jax.ShapeDtypeStruct((2, N_S, 128), _F32),
        mesh=_sc_mesh(),
        scratch_types=[
            pltpu.VMEM((CPW, CHUNK), jnp.int32),
            pltpu.VMEM((CHUNK, 128), _F32),
            pltpu.VMEM((200, 128), _F32),
            pltpu.VMEM_SHARED((N_S, 128), _F32),
            pltpu.SemaphoreType.DMA,
        ],
    )(e1r, msg128, zer200)


def kernel(x, pos, batch, x_skip, pos_skip, batch_skip, edge_index,
           global_token, Wq, bq, Wk, bk, Wv, bv, in_proj_w, in_proj_b,
           out_proj_w, out_proj_b, mlp_W, mlp_b, bn_g, bn_b, root_W, conv_b):
    f32 = _F32
    posT_pad = jnp.pad(pos.T, ((0, 5), (0, NCP - N_C)),
                       constant_values=1e12)
    x_pad = jnp.pad(x, ((0, NCP - N_C), (0, 0)))
    pos_pad = jnp.pad(pos_skip, ((0, 0), (0, 5)))
    e0p = jnp.pad(edge_index[0], (0, EPAD - E))
    e1p = jnp.pad(edge_index[1], (0, EPAD - E))
    e0r = e0p.reshape(EPAD // CHUNK, CHUNK)
    e1r = e1p.reshape(EPAD // CHUNK, CHUNK)
    e0g = e0p.reshape(EPAD // CHUNKG, CHUNKG)
    e1g = e1p.reshape(EPAD // CHUNKG, CHUNKG)
    wq_pad = jnp.pad(Wq, ((0, 0), (0, 5)))
    bias = jnp.stack([bq, bk, bv, in_proj_b[:EMB], in_proj_b[EMB:2 * EMB],
                      in_proj_b[2 * EMB:], out_proj_b,
                      jnp.zeros((EMB,), f32)])
    pe = jnp.asarray(_PE)
    wpad = jnp.pad(mlp_W, ((0, 0), (0, 0), (0, 128 - 67)))
    perm = jnp.asarray(_PERM)
    rt = jnp.asarray(_RT)
    rm = jnp.asarray(_RM)

    def node_table(h_):
        return jnp.pad(jnp.concatenate([pos_pad, h_], axis=1),
                       ((0, 0), (0, 128 - 24)))

    h0 = pl.pallas_call(
        _knn_body,
        grid=(N_S // TS,),
        in_specs=[
            pl.BlockSpec((TS, 3), lambda i: (i, 0)),
            pl.BlockSpec((8, NCP), lambda i: (0, 0)),
            pl.BlockSpec((NCP, 8), lambda i: (0, 0)),
            pl.BlockSpec((TS, 8), lambda i: (i, 0)),
        ],
        out_specs=pl.BlockSpec((TS, 16), lambda i: (i, 0)),
        out_shape=jax.ShapeDtypeStruct((N_S, 16), f32),
    )(pos_skip, posT_pad, x_pad, x_skip)

    ntab0 = node_table(h0)
    g0e = _run_sc_gone(e0g, ntab0)
    g1e = _run_sc_gone(e1g, ntab0)
    ones128 = jnp.pad(jnp.ones((CHUNK, 16), f32), ((0, 0), (0, 112)))
    zer200 = jnp.zeros((200, 128), f32)
    cntp = _run_sc_cnt(e1r, ones128, zer200)

    kT, vproj, qwT, cons, woT = pl.pallas_call(
        _prologue_body,
        grid=(1,),
        in_specs=[pl.BlockSpec(s, lambda i: (0, 0))
                  for s in ((T, 1024), (EMB, 1024), (EMB, 1024), (192, EMB),
                            (EMB, 8), (EMB, EMB), (8, EMB), (T, EMB))],
        out_specs=[
            pl.BlockSpec((EMB, T), lambda i: (0, 0)),
            pl.BlockSpec((T, EMB), lambda i: (0, 0)),
            pl.BlockSpec((8, EMB), lambda i: (0, 0)),
            pl.BlockSpec((8, EMB), lambda i: (0, 0)),
            pl.BlockSpec((EMB, EMB), lambda i: (0, 0)),
        ],
        out_shape=[
            jax.ShapeDtypeStruct((EMB, T), f32),
            jax.ShapeDtypeStruct((T, EMB), f32),
            jax.ShapeDtypeStruct((8, EMB), f32),
            jax.ShapeDtypeStruct((8, EMB), f32),
            jax.ShapeDtypeStruct((EMB, EMB), f32),
        ],
    )(global_token, Wk, Wv, in_proj_w, wq_pad, out_proj_w, bias, pe)

    gtok, sp, s1, s2 = pl.pallas_call(
        _attn_body,
        grid=(E // TE,),
        in_specs=[
            pl.BlockSpec((TE, 128), lambda i: (i, 0)),
            pl.BlockSpec((TE, 128), lambda i: (i, 0)),
            pl.BlockSpec((EMB, T), lambda i: (0, 0)),
            pl.BlockSpec((T, EMB), lambda i: (0, 0)),
            pl.BlockSpec((8, EMB), lambda i: (0, 0)),
            pl.BlockSpec((8, EMB), lambda i: (0, 0)),
            pl.BlockSpec((EMB, EMB), lambda i: (0, 0)),
        ],
        out_specs=[
            pl.BlockSpec((TE, EMB), lambda i: (i, 0)),
            pl.BlockSpec((TE, 8), lambda i: (i, 0)),
            pl.BlockSpec((8, 128), lambda i: (0, 0)),
            pl.BlockSpec((128, 128), lambda i: (0, 0)),
        ],
        out_shape=[
            jax.ShapeDtypeStruct((E, EMB), f32),
            jax.ShapeDtypeStruct((E, 8), f32),
            jax.ShapeDtypeStruct((8, 128), f32),
            jax.ShapeDtypeStruct((128, 128), f32),
        ],
    )(g0e, g1e, kT, vproj, qwT, cons, woT)

    ap_all, bp_all = pl.pallas_call(
        _fold_body,
        grid=(1,),
        in_specs=[
            pl.BlockSpec((8, 128), lambda i: (0, 0)),
            pl.BlockSpec((128, 128), lambda i: (0, 0)),
            pl.BlockSpec((3, 256, 128), lambda i: (0, 0, 0)),
            pl.BlockSpec((256, 3), lambda i: (0, 0)),
            pl.BlockSpec((256, 3), lambda i: (0, 0)),
            pl.BlockSpec((256, 256), lambda i: (0, 0)),
        ],
        out_specs=[
            pl.BlockSpec((3, 256, 128), lambda i: (0, 0, 0)),
            pl.BlockSpec((3, 8, 256), lambda i: (0, 0, 0)),
        ],
        out_shape=[
            jax.ShapeDtypeStruct((3, 256, 128), f32),
            jax.ShapeDtypeStruct((3, 8, 256), f32),
        ],
    )(s1, s2, wpad, bn_g.T, bn_b.T, perm)

    h = h0
    hg = g0e
    for l in range(3):
        msg = pl.pallas_call(
            _edge_body,
            grid=(E // TE,),
            in_specs=[
                pl.BlockSpec((TE, 8), lambda i: (i, 0)),
                pl.BlockSpec((TE, EMB), lambda i: (i, 0)),
                pl.BlockSpec((TE, 128), lambda i: (i, 0)),
                pl.BlockSpec((256, 128), lambda i: (0, 0)),
                pl.BlockSpec((8, 256), lambda i: (0, 0)),
                pl.BlockSpec((16, 256), lambda i: (0, 0)),
                pl.BlockSpec((256, 16), lambda i: (0, 0)),
            ],
            out_specs=pl.BlockSpec((TE, 128), lambda i: (i, 0)),
            out_shape=jax.ShapeDtypeStruct((E, 128), f32),
        )(sp, gtok, hg, ap_all[l], bp_all[l], rt, rm)

        aggp = _run_sc_scat(e1r, msg, zer200)

        h = pl.pallas_call(
            lambda *a, l=l: _update_body(*a, l=l),
            grid=(1,),
            in_specs=[
                pl.BlockSpec((N_S, 16), lambda i: (0, 0)),
                pl.BlockSpec((2, N_S, 128), lambda i: (0, 0, 0)),
                pl.BlockSpec((2, N_S, 128), lambda i: (0, 0, 0)),
                pl.BlockSpec((3, 16, 16), lambda i: (0, 0, 0)),
                pl.BlockSpec((3, 16), lambda i: (0, 0)),
            ],
            out_specs=pl.BlockSpec((N_S, 16), lambda i: (0, 0)),
            out_shape=jax.ShapeDtypeStruct((N_S, 16), f32),
        )(h, aggp, cntp, root_W, conv_b)

        if l < 2:
            hg = _run_sc_gone(e0g, node_table(h))

    return (h, pos_skip, batch_skip)
